# Initial kernel scaffold; baseline (speedup 1.0000x reference)
import jax
import jax.numpy as jnp
from jax.experimental import pallas as pl


def kernel(hidden, src, attn, W, b, alignment):
    raise NotImplementedError("write your pallas kernel here")


# speedup vs baseline: 9.4303x; 9.4303x over previous
import functools

import jax
import jax.numpy as jnp
from jax import lax
from jax.experimental import pallas as pl

_SRC_VOCAB = 50000
_RNN = 512
_SRC_LEN = 128
_BATCH = 32
_TGT_LEN = 32
_COPY_COL = 4
_PAD_COL = 0
_ROWS = _TGT_LEN * _BATCH

_TV = 1024


def _pass1_body(nv, hidden_ref, w_ref, b_ref, attn_ref, s2t_ref,
                logits_ref, m_ref, s_ref, copyl_ref, padl_ref, norm_ref):
    v = pl.program_id(0)
    h = hidden_ref[...]
    l = lax.dot_general(h, w_ref[...], (((1,), (1,)), ((), ())),
                        preferred_element_type=jnp.float32,
                        precision=lax.Precision.HIGHEST)
    l = l + b_ref[...]
    gcol = v * _TV + lax.broadcasted_iota(jnp.int32, (1, _TV), 1)
    l2 = jnp.where(gcol == _COPY_COL, jnp.float32(1e-10), l)
    logits_ref[...] = l2
    tile_m = jnp.max(l2, axis=1, keepdims=True)

    @pl.when(v == 0)
    def _():
        m_ref[...] = tile_m
        s_ref[...] = jnp.sum(jnp.exp(l2 - tile_m), axis=1, keepdims=True)
        copyl_ref[...] = l[:, _COPY_COL:_COPY_COL + 1]
        padl_ref[...] = l2[:, _PAD_COL:_PAD_COL + 1]

    @pl.when(v > 0)
    def _():
        m_old = m_ref[...]
        m_new = jnp.maximum(m_old, tile_m)
        s_ref[...] = (s_ref[...] * jnp.exp(m_old - m_new)
                      + jnp.sum(jnp.exp(l2 - m_new), axis=1, keepdims=True))
        m_ref[...] = m_new

    @pl.when(v == nv - 1)
    def _():
        copy = jax.nn.sigmoid(copyl_ref[...])
        smpad = jnp.exp(padl_ref[...] - m_ref[...]) / s_ref[...]
        attn3 = attn_ref[...].reshape(_TGT_LEN, _BATCH, _SRC_LEN)
        padmask = (s2t_ref[...] == _PAD_COL).astype(jnp.float32)
        attn_sum = jnp.sum(attn3, axis=2).reshape(_ROWS, 1)
        pad_sum = jnp.sum(attn3 * padmask[None, :, :], axis=2).reshape(_ROWS, 1)
        norm_ref[...] = ((1.0 - copy) * (1.0 - smpad)
                         + copy * (attn_sum - pad_sum) + 1e-10)


def _pass2_body(logits_ref, m_ref, s_ref, copyl_ref, norm_ref, attn_ref,
                s2t_ref, out_ref):
    v = pl.program_id(0)
    copy = jax.nn.sigmoid(copyl_ref[...])
    p = (jnp.exp(logits_ref[...] - m_ref[...]) / s_ref[...]) * (1.0 - copy)
    attnw3 = (attn_ref[...] * copy).reshape(_TGT_LEN, _BATCH, _SRC_LEN)
    base = v * _TV
    cols = base + lax.broadcasted_iota(jnp.int32, (_SRC_LEN, _TV), 1)
    parts = []
    for b in range(_BATCH):
        idxb = s2t_ref[b, :]
        oh = (idxb[:, None] == cols).astype(jnp.float32)
        sc_b = jnp.dot(attnw3[:, b, :], oh,
                       preferred_element_type=jnp.float32)
        parts.append(sc_b[:, None, :])
    sc = jnp.concatenate(parts, axis=1).reshape(_ROWS, _TV)
    x = p + sc
    gcol = base + lax.broadcasted_iota(jnp.int32, (1, _TV), 1)
    x = jnp.where(gcol == _PAD_COL, jnp.float32(1e-10), x)
    out_ref[...] = jnp.log(x / norm_ref[...] + 1e-10)


@jax.jit
def kernel(hidden, src, attn, W, b, alignment):
    tgt_vocab = W.shape[0]
    nv = (tgt_vocab + _TV - 1) // _TV
    vpad = nv * _TV
    w_p = jnp.pad(W, ((0, vpad - tgt_vocab), (0, 0)))
    b_p = jnp.pad(b, (0, vpad - tgt_vocab), constant_values=-1e30)[None, :]

    words = src[:, :, 0]
    s2t = jnp.take(alignment, words.reshape(-1), axis=0)
    s2t = s2t.reshape(_SRC_LEN, _BATCH).T

    row = pl.BlockSpec((_ROWS, 1), lambda v: (0, 0))
    logits, m, s, copyl, padl, norm = pl.pallas_call(
        functools.partial(_pass1_body, nv),
        grid=(nv,),
        in_specs=[
            pl.BlockSpec((_ROWS, _RNN), lambda v: (0, 0)),
            pl.BlockSpec((_TV, _RNN), lambda v: (v, 0)),
            pl.BlockSpec((1, _TV), lambda v: (0, v)),
            pl.BlockSpec((_ROWS, _SRC_LEN), lambda v: (0, 0)),
            pl.BlockSpec((_BATCH, _SRC_LEN), lambda v: (0, 0)),
        ],
        out_specs=[
            pl.BlockSpec((_ROWS, _TV), lambda v: (0, v)),
            row, row, row, row, row,
        ],
        out_shape=[
            jax.ShapeDtypeStruct((_ROWS, vpad), jnp.float32),
            jax.ShapeDtypeStruct((_ROWS, 1), jnp.float32),
            jax.ShapeDtypeStruct((_ROWS, 1), jnp.float32),
            jax.ShapeDtypeStruct((_ROWS, 1), jnp.float32),
            jax.ShapeDtypeStruct((_ROWS, 1), jnp.float32),
            jax.ShapeDtypeStruct((_ROWS, 1), jnp.float32),
        ],
    )(hidden, w_p, b_p, attn, s2t)

    out = pl.pallas_call(
        _pass2_body,
        grid=(nv,),
        in_specs=[
            pl.BlockSpec((_ROWS, _TV), lambda v: (0, v)),
            row, row, row, row,
            pl.BlockSpec((_ROWS, _SRC_LEN), lambda v: (0, 0)),
            pl.BlockSpec((_BATCH, _SRC_LEN), lambda v: (0, 0)),
        ],
        out_specs=pl.BlockSpec((_ROWS, _TV), lambda v: (0, v)),
        out_shape=jax.ShapeDtypeStruct((_ROWS, vpad), jnp.float32),
    )(logits, m, s, copyl, norm, attn, s2t)

    return out[:, :tgt_vocab]


# speedup vs baseline: 12.3257x; 1.3070x over previous
import functools

import jax
import jax.numpy as jnp
from jax import lax
from jax.experimental import pallas as pl

_SRC_VOCAB = 50000
_RNN = 512
_SRC_LEN = 128
_BATCH = 32
_TGT_LEN = 32
_COPY_COL = 4
_PAD_COL = 0
_ROWS = _TGT_LEN * _BATCH

_TV = 1024


def _pass1_body(nv, hidden_ref, w_ref, b_ref, attn_ref, s2t_ref,
                logits_ref, m_ref, s_ref, copyl_ref, padl_ref, norm_ref):
    v = pl.program_id(0)
    h = hidden_ref[...]
    l = lax.dot_general(h, w_ref[...], (((1,), (1,)), ((), ())),
                        preferred_element_type=jnp.float32,
                        precision=lax.Precision.DEFAULT)
    l = l + b_ref[...]
    gcol = v * _TV + lax.broadcasted_iota(jnp.int32, (1, _TV), 1)
    l2 = jnp.where(gcol == _COPY_COL, jnp.float32(1e-10), l)
    logits_ref[...] = l2.astype(jnp.bfloat16)
    tile_m = jnp.max(l2, axis=1, keepdims=True)

    @pl.when(v == 0)
    def _():
        m_ref[...] = tile_m
        s_ref[...] = jnp.sum(jnp.exp(l2 - tile_m), axis=1, keepdims=True)
        copyl_ref[...] = l[:, _COPY_COL:_COPY_COL + 1]
        padl_ref[...] = l2[:, _PAD_COL:_PAD_COL + 1]

    @pl.when(v > 0)
    def _():
        m_old = m_ref[...]
        m_new = jnp.maximum(m_old, tile_m)
        s_ref[...] = (s_ref[...] * jnp.exp(m_old - m_new)
                      + jnp.sum(jnp.exp(l2 - m_new), axis=1, keepdims=True))
        m_ref[...] = m_new

    @pl.when(v == nv - 1)
    def _():
        copy = jax.nn.sigmoid(copyl_ref[...])
        smpad = jnp.exp(padl_ref[...] - m_ref[...]) / s_ref[...]
        attn3 = attn_ref[...].reshape(_TGT_LEN, _BATCH, _SRC_LEN)
        padmask = (s2t_ref[...] == _PAD_COL).astype(jnp.float32)
        attn_sum = jnp.sum(attn3, axis=2).reshape(_ROWS, 1)
        pad_sum = jnp.sum(attn3 * padmask[None, :, :], axis=2).reshape(_ROWS, 1)
        norm_ref[...] = ((1.0 - copy) * (1.0 - smpad)
                         + copy * (attn_sum - pad_sum) + 1e-10)


def _pass2_body(logits_ref, m_ref, s_ref, copyl_ref, norm_ref, attn_ref,
                s2t_ref, out_ref):
    v = pl.program_id(0)
    copy = jax.nn.sigmoid(copyl_ref[...])
    lt = logits_ref[...].astype(jnp.float32)
    p = (jnp.exp(lt - m_ref[...]) / s_ref[...]) * (1.0 - copy)
    attnw3 = (attn_ref[...] * copy).reshape(_TGT_LEN, _BATCH, _SRC_LEN)
    base = v * _TV
    cols = base + lax.broadcasted_iota(jnp.int32, (_SRC_LEN, _TV), 1)
    parts = []
    for b in range(_BATCH):
        idxb = s2t_ref[b, :]
        oh = (idxb[:, None] == cols).astype(jnp.float32)
        sc_b = jnp.dot(attnw3[:, b, :], oh,
                       preferred_element_type=jnp.float32)
        parts.append(sc_b[:, None, :])
    sc = jnp.concatenate(parts, axis=1).reshape(_ROWS, _TV)
    x = p + sc
    gcol = base + lax.broadcasted_iota(jnp.int32, (1, _TV), 1)
    x = jnp.where(gcol == _PAD_COL, jnp.float32(1e-10), x)
    out_ref[...] = jnp.log(x / norm_ref[...] + 1e-10)


@jax.jit
def kernel(hidden, src, attn, W, b, alignment):
    tgt_vocab = W.shape[0]
    nv = (tgt_vocab + _TV - 1) // _TV
    vpad = nv * _TV
    w_p = jnp.pad(W, ((0, vpad - tgt_vocab), (0, 0)))
    b_p = jnp.pad(b, (0, vpad - tgt_vocab), constant_values=-1e30)[None, :]

    words = src[:, :, 0]
    s2t = jnp.take(alignment, words.reshape(-1), axis=0)
    s2t = s2t.reshape(_SRC_LEN, _BATCH).T

    row = pl.BlockSpec((_ROWS, 1), lambda v: (0, 0))
    logits, m, s, copyl, padl, norm = pl.pallas_call(
        functools.partial(_pass1_body, nv),
        grid=(nv,),
        in_specs=[
            pl.BlockSpec((_ROWS, _RNN), lambda v: (0, 0)),
            pl.BlockSpec((_TV, _RNN), lambda v: (v, 0)),
            pl.BlockSpec((1, _TV), lambda v: (0, v)),
            pl.BlockSpec((_ROWS, _SRC_LEN), lambda v: (0, 0)),
            pl.BlockSpec((_BATCH, _SRC_LEN), lambda v: (0, 0)),
        ],
        out_specs=[
            pl.BlockSpec((_ROWS, _TV), lambda v: (0, v)),
            row, row, row, row, row,
        ],
        out_shape=[
            jax.ShapeDtypeStruct((_ROWS, vpad), jnp.bfloat16),
            jax.ShapeDtypeStruct((_ROWS, 1), jnp.float32),
            jax.ShapeDtypeStruct((_ROWS, 1), jnp.float32),
            jax.ShapeDtypeStruct((_ROWS, 1), jnp.float32),
            jax.ShapeDtypeStruct((_ROWS, 1), jnp.float32),
            jax.ShapeDtypeStruct((_ROWS, 1), jnp.float32),
        ],
    )(hidden, w_p, b_p, attn, s2t)

    out = pl.pallas_call(
        _pass2_body,
        grid=(nv,),
        in_specs=[
            pl.BlockSpec((_ROWS, _TV), lambda v: (0, v)),
            row, row, row, row,
            pl.BlockSpec((_ROWS, _SRC_LEN), lambda v: (0, 0)),
            pl.BlockSpec((_BATCH, _SRC_LEN), lambda v: (0, 0)),
        ],
        out_specs=pl.BlockSpec((_ROWS, _TV), lambda v: (0, v)),
        out_shape=jax.ShapeDtypeStruct((_ROWS, vpad), jnp.float32),
    )(logits, m, s, copyl, norm, attn, s2t)

    return out[:, :tgt_vocab]


# speedup vs baseline: 14.8743x; 1.2068x over previous
import functools

import jax
import jax.numpy as jnp
from jax import lax
from jax.experimental import pallas as pl

_SRC_VOCAB = 50000
_RNN = 512
_SRC_LEN = 128
_BATCH = 32
_TGT_LEN = 32
_COPY_COL = 4
_PAD_COL = 0
_ROWS = _TGT_LEN * _BATCH

_TV = 1024


def _pass1_body(nv, vocab, hidden_ref, w_ref, b_ref, attn_ref, s2t_ref,
                logits_ref, m_ref, s_ref, copyl_ref, padl_ref, norm_ref):
    v = pl.program_id(0)
    h = hidden_ref[...]
    l = lax.dot_general(h, w_ref[...], (((1,), (1,)), ((), ())),
                        preferred_element_type=jnp.float32,
                        precision=lax.Precision.DEFAULT)
    l = l + b_ref[...]
    gcol = v * _TV + lax.broadcasted_iota(jnp.int32, (1, _TV), 1)
    l2 = jnp.where(gcol == _COPY_COL, jnp.float32(1e-10), l)
    l2 = jnp.where(gcol < vocab, l2, jnp.float32(-1e30))
    logits_ref[...] = l2.astype(jnp.bfloat16)
    tile_m = jnp.max(l2, axis=1, keepdims=True)

    @pl.when(v == 0)
    def _():
        m_ref[...] = tile_m
        s_ref[...] = jnp.sum(jnp.exp(l2 - tile_m), axis=1, keepdims=True)
        copyl_ref[...] = l[:, _COPY_COL:_COPY_COL + 1]
        padl_ref[...] = l2[:, _PAD_COL:_PAD_COL + 1]

    @pl.when(v > 0)
    def _():
        m_old = m_ref[...]
        m_new = jnp.maximum(m_old, tile_m)
        s_ref[...] = (s_ref[...] * jnp.exp(m_old - m_new)
                      + jnp.sum(jnp.exp(l2 - m_new), axis=1, keepdims=True))
        m_ref[...] = m_new

    @pl.when(v == nv - 1)
    def _():
        copy = jax.nn.sigmoid(copyl_ref[...])
        smpad = jnp.exp(padl_ref[...] - m_ref[...]) / s_ref[...]
        attn3 = attn_ref[...].reshape(_TGT_LEN, _BATCH, _SRC_LEN)
        padmask = (s2t_ref[...] == _PAD_COL).astype(jnp.float32)
        attn_sum = jnp.sum(attn3, axis=2).reshape(_ROWS, 1)
        pad_sum = jnp.sum(attn3 * padmask[None, :, :], axis=2).reshape(_ROWS, 1)
        norm_ref[...] = ((1.0 - copy) * (1.0 - smpad)
                         + copy * (attn_sum - pad_sum) + 1e-10)


def _pass2_body(logits_ref, m_ref, s_ref, copyl_ref, norm_ref, attn_ref,
                s2t_ref, out_ref):
    v = pl.program_id(0)
    copy = jax.nn.sigmoid(copyl_ref[...])
    lt = logits_ref[...].astype(jnp.float32)
    p = (jnp.exp(lt - m_ref[...]) / s_ref[...]) * (1.0 - copy)
    attnw3 = (attn_ref[...] * copy).reshape(_TGT_LEN, _BATCH, _SRC_LEN)
    base = v * _TV
    cols = base + lax.broadcasted_iota(jnp.int32, (_SRC_LEN, _TV), 1)
    parts = []
    for b in range(_BATCH):
        idxb = s2t_ref[b, :]
        oh = (idxb[:, None] == cols).astype(jnp.float32)
        sc_b = jnp.dot(attnw3[:, b, :], oh,
                       preferred_element_type=jnp.float32)
        parts.append(sc_b[:, None, :])
    sc = jnp.concatenate(parts, axis=1).reshape(_ROWS, _TV)
    x = p + sc
    gcol = base + lax.broadcasted_iota(jnp.int32, (1, _TV), 1)
    x = jnp.where(gcol == _PAD_COL, jnp.float32(1e-10), x)
    out_ref[...] = jnp.log(x / norm_ref[...] + 1e-10)


@jax.jit
def kernel(hidden, src, attn, W, b, alignment):
    tgt_vocab = W.shape[0]
    nv = (tgt_vocab + _TV - 1) // _TV
    w_p = W
    b_p = b[None, :]

    words = src[:, :, 0]
    s2t = jnp.take(alignment, words.reshape(-1), axis=0)
    s2t = s2t.reshape(_SRC_LEN, _BATCH).T

    row = pl.BlockSpec((_ROWS, 1), lambda v: (0, 0))
    logits, m, s, copyl, padl, norm = pl.pallas_call(
        functools.partial(_pass1_body, nv, tgt_vocab),
        grid=(nv,),
        in_specs=[
            pl.BlockSpec((_ROWS, _RNN), lambda v: (0, 0)),
            pl.BlockSpec((_TV, _RNN), lambda v: (v, 0)),
            pl.BlockSpec((1, _TV), lambda v: (0, v)),
            pl.BlockSpec((_ROWS, _SRC_LEN), lambda v: (0, 0)),
            pl.BlockSpec((_BATCH, _SRC_LEN), lambda v: (0, 0)),
        ],
        out_specs=[
            pl.BlockSpec((_ROWS, _TV), lambda v: (0, v)),
            row, row, row, row, row,
        ],
        out_shape=[
            jax.ShapeDtypeStruct((_ROWS, tgt_vocab), jnp.bfloat16),
            jax.ShapeDtypeStruct((_ROWS, 1), jnp.float32),
            jax.ShapeDtypeStruct((_ROWS, 1), jnp.float32),
            jax.ShapeDtypeStruct((_ROWS, 1), jnp.float32),
            jax.ShapeDtypeStruct((_ROWS, 1), jnp.float32),
            jax.ShapeDtypeStruct((_ROWS, 1), jnp.float32),
        ],
    )(hidden, w_p, b_p, attn, s2t)

    out = pl.pallas_call(
        _pass2_body,
        grid=(nv,),
        in_specs=[
            pl.BlockSpec((_ROWS, _TV), lambda v: (0, v)),
            row, row, row, row,
            pl.BlockSpec((_ROWS, _SRC_LEN), lambda v: (0, 0)),
            pl.BlockSpec((_BATCH, _SRC_LEN), lambda v: (0, 0)),
        ],
        out_specs=pl.BlockSpec((_ROWS, _TV), lambda v: (0, v)),
        out_shape=jax.ShapeDtypeStruct((_ROWS, tgt_vocab), jnp.float32),
    )(logits, m, s, copyl, norm, attn, s2t)

    return out


# speedup vs baseline: 14.9528x; 1.0053x over previous
import functools

import jax
import jax.numpy as jnp
from jax import lax
from jax.experimental import pallas as pl
from jax.experimental.pallas import tpu as pltpu
from jax.experimental.pallas import tpu_sc as plsc

_SRC_VOCAB = 50000
_RNN = 512
_SRC_LEN = 128
_BATCH = 32
_TGT_LEN = 32
_COPY_COL = 4
_PAD_COL = 0
_ROWS = _TGT_LEN * _BATCH

_TV = 1024


_SC_NC = 2
_SC_NS = 16
_SC_NW = _SC_NC * _SC_NS
_SC_CHUNK = (_SRC_LEN * _BATCH) // _SC_NW


def _sc_gather_body(table_hbm, idx_hbm, out_hbm, idx_v, rows_v, sem):
    wid = lax.axis_index("s") * _SC_NC + lax.axis_index("c")
    base = wid * _SC_CHUNK
    pltpu.sync_copy(idx_hbm.at[pl.ds(base, _SC_CHUNK)], idx_v)
    pltpu.async_copy(table_hbm.at[idx_v], rows_v, sem).wait()
    pltpu.sync_copy(rows_v, out_hbm.at[pl.ds(base, _SC_CHUNK)])


def _sc_gather(table, idx_flat):
    return pl.kernel(
        _sc_gather_body,
        mesh=plsc.VectorSubcoreMesh(core_axis_name="c", subcore_axis_name="s"),
        out_type=jax.ShapeDtypeStruct((_SRC_LEN * _BATCH,), jnp.int32),
        scratch_types=[
            pltpu.VMEM((_SC_CHUNK,), jnp.int32),
            pltpu.VMEM((_SC_CHUNK,), jnp.int32),
            pltpu.SemaphoreType.DMA,
        ],
    )(table, idx_flat)


def _pass1_body(nv, vocab, hidden_ref, w_ref, b_ref, attn_ref, s2t_ref,
                logits_ref, m_ref, s_ref, copyl_ref, padl_ref, norm_ref):
    v = pl.program_id(0)
    h = hidden_ref[...]
    l = lax.dot_general(h, w_ref[...], (((1,), (1,)), ((), ())),
                        preferred_element_type=jnp.float32,
                        precision=lax.Precision.DEFAULT)
    l = l + b_ref[...]
    gcol = v * _TV + lax.broadcasted_iota(jnp.int32, (1, _TV), 1)
    l2 = jnp.where(gcol == _COPY_COL, jnp.float32(1e-10), l)
    l2 = jnp.where(gcol < vocab, l2, jnp.float32(-1e30))
    logits_ref[...] = l2.astype(jnp.bfloat16)
    tile_m = jnp.max(l2, axis=1, keepdims=True)

    @pl.when(v == 0)
    def _():
        m_ref[...] = tile_m
        s_ref[...] = jnp.sum(jnp.exp(l2 - tile_m), axis=1, keepdims=True)
        copyl_ref[...] = l[:, _COPY_COL:_COPY_COL + 1]
        padl_ref[...] = l2[:, _PAD_COL:_PAD_COL + 1]

    @pl.when(v > 0)
    def _():
        m_old = m_ref[...]
        m_new = jnp.maximum(m_old, tile_m)
        s_ref[...] = (s_ref[...] * jnp.exp(m_old - m_new)
                      + jnp.sum(jnp.exp(l2 - m_new), axis=1, keepdims=True))
        m_ref[...] = m_new

    @pl.when(v == nv - 1)
    def _():
        copy = jax.nn.sigmoid(copyl_ref[...])
        smpad = jnp.exp(padl_ref[...] - m_ref[...]) / s_ref[...]
        attn3 = attn_ref[...].reshape(_TGT_LEN, _BATCH, _SRC_LEN)
        padmask = (s2t_ref[...] == _PAD_COL).astype(jnp.float32)
        attn_sum = jnp.sum(attn3, axis=2).reshape(_ROWS, 1)
        pad_sum = jnp.sum(attn3 * padmask[None, :, :], axis=2).reshape(_ROWS, 1)
        norm_ref[...] = ((1.0 - copy) * (1.0 - smpad)
                         + copy * (attn_sum - pad_sum) + 1e-10)


def _pass2_body(logits_ref, m_ref, s_ref, copyl_ref, norm_ref, attn_ref,
                s2t_ref, out_ref):
    v = pl.program_id(0)
    copy = jax.nn.sigmoid(copyl_ref[...])
    lt = logits_ref[...].astype(jnp.float32)
    p = (jnp.exp(lt - m_ref[...]) / s_ref[...]) * (1.0 - copy)
    attnw3 = (attn_ref[...] * copy).reshape(_TGT_LEN, _BATCH, _SRC_LEN)
    base = v * _TV
    cols = base + lax.broadcasted_iota(jnp.int32, (_SRC_LEN, _TV), 1)
    parts = []
    for b in range(_BATCH):
        idxb = s2t_ref[b, :]
        oh = (idxb[:, None] == cols).astype(jnp.float32)
        sc_b = jnp.dot(attnw3[:, b, :], oh,
                       preferred_element_type=jnp.float32)
        parts.append(sc_b[:, None, :])
    sc = jnp.concatenate(parts, axis=1).reshape(_ROWS, _TV)
    x = p + sc
    gcol = base + lax.broadcasted_iota(jnp.int32, (1, _TV), 1)
    x = jnp.where(gcol == _PAD_COL, jnp.float32(1e-10), x)
    out_ref[...] = jnp.log(x / norm_ref[...] + 1e-10)


@jax.jit
def kernel(hidden, src, attn, W, b, alignment):
    tgt_vocab = W.shape[0]
    nv = (tgt_vocab + _TV - 1) // _TV
    w_p = W
    b_p = b[None, :]

    words = src[:, :, 0]
    s2t = _sc_gather(alignment, words.T.reshape(-1))
    s2t = s2t.reshape(_BATCH, _SRC_LEN)

    row = pl.BlockSpec((_ROWS, 1), lambda v: (0, 0))
    logits, m, s, copyl, padl, norm = pl.pallas_call(
        functools.partial(_pass1_body, nv, tgt_vocab),
        grid=(nv,),
        in_specs=[
            pl.BlockSpec((_ROWS, _RNN), lambda v: (0, 0)),
            pl.BlockSpec((_TV, _RNN), lambda v: (v, 0)),
            pl.BlockSpec((1, _TV), lambda v: (0, v)),
            pl.BlockSpec((_ROWS, _SRC_LEN), lambda v: (0, 0)),
            pl.BlockSpec((_BATCH, _SRC_LEN), lambda v: (0, 0)),
        ],
        out_specs=[
            pl.BlockSpec((_ROWS, _TV), lambda v: (0, v)),
            row, row, row, row, row,
        ],
        out_shape=[
            jax.ShapeDtypeStruct((_ROWS, tgt_vocab), jnp.bfloat16),
            jax.ShapeDtypeStruct((_ROWS, 1), jnp.float32),
            jax.ShapeDtypeStruct((_ROWS, 1), jnp.float32),
            jax.ShapeDtypeStruct((_ROWS, 1), jnp.float32),
            jax.ShapeDtypeStruct((_ROWS, 1), jnp.float32),
            jax.ShapeDtypeStruct((_ROWS, 1), jnp.float32),
        ],
    )(hidden, w_p, b_p, attn, s2t)

    out = pl.pallas_call(
        _pass2_body,
        grid=(nv,),
        in_specs=[
            pl.BlockSpec((_ROWS, _TV), lambda v: (0, v)),
            row, row, row, row,
            pl.BlockSpec((_ROWS, _SRC_LEN), lambda v: (0, 0)),
            pl.BlockSpec((_BATCH, _SRC_LEN), lambda v: (0, 0)),
        ],
        out_specs=pl.BlockSpec((_ROWS, _TV), lambda v: (0, v)),
        out_shape=jax.ShapeDtypeStruct((_ROWS, tgt_vocab), jnp.float32),
    )(logits, m, s, copyl, norm, attn, s2t)

    return out


# speedup vs baseline: 15.4023x; 1.0301x over previous
import functools

import jax
import jax.numpy as jnp
from jax import lax
from jax.experimental import pallas as pl
from jax.experimental.pallas import tpu as pltpu
from jax.experimental.pallas import tpu_sc as plsc

_SRC_VOCAB = 50000
_RNN = 512
_SRC_LEN = 128
_BATCH = 32
_TGT_LEN = 32
_COPY_COL = 4
_PAD_COL = 0
_ROWS = _TGT_LEN * _BATCH

_TV = 2048


_SC_NC = 2
_SC_NS = 16
_SC_NW = _SC_NC * _SC_NS
_SC_CHUNK = (_SRC_LEN * _BATCH) // _SC_NW


def _sc_gather_body(table_hbm, idx_hbm, out_hbm, idx_v, rows_v, sem):
    wid = lax.axis_index("s") * _SC_NC + lax.axis_index("c")
    base = wid * _SC_CHUNK
    pltpu.sync_copy(idx_hbm.at[pl.ds(base, _SC_CHUNK)], idx_v)
    pltpu.async_copy(table_hbm.at[idx_v], rows_v, sem).wait()
    pltpu.sync_copy(rows_v, out_hbm.at[pl.ds(base, _SC_CHUNK)])


def _sc_gather(table, idx_flat):
    return pl.kernel(
        _sc_gather_body,
        mesh=plsc.VectorSubcoreMesh(core_axis_name="c", subcore_axis_name="s"),
        out_type=jax.ShapeDtypeStruct((_SRC_LEN * _BATCH,), jnp.int32),
        scratch_types=[
            pltpu.VMEM((_SC_CHUNK,), jnp.int32),
            pltpu.VMEM((_SC_CHUNK,), jnp.int32),
            pltpu.SemaphoreType.DMA,
        ],
    )(table, idx_flat)


def _pass1_body(nv, vocab, hidden_ref, w_ref, b_ref, attn_ref, s2t_ref,
                logits_ref, m_ref, s_ref, copyl_ref, padl_ref, norm_ref):
    v = pl.program_id(0)
    h = hidden_ref[...]
    l = lax.dot_general(h, w_ref[...], (((1,), (1,)), ((), ())),
                        preferred_element_type=jnp.float32,
                        precision=lax.Precision.DEFAULT)
    l = l + b_ref[...]
    gcol = v * _TV + lax.broadcasted_iota(jnp.int32, (1, _TV), 1)
    l2 = jnp.where(gcol == _COPY_COL, jnp.float32(1e-10), l)
    l2 = jnp.where(gcol < vocab, l2, jnp.float32(-1e30))
    logits_ref[...] = l2.astype(jnp.bfloat16)
    tile_m = jnp.max(l2, axis=1, keepdims=True)

    @pl.when(v == 0)
    def _():
        m_ref[...] = tile_m
        s_ref[...] = jnp.sum(jnp.exp(l2 - tile_m), axis=1, keepdims=True)
        copyl_ref[...] = l[:, _COPY_COL:_COPY_COL + 1]
        padl_ref[...] = l2[:, _PAD_COL:_PAD_COL + 1]

    @pl.when(v > 0)
    def _():
        m_old = m_ref[...]
        m_new = jnp.maximum(m_old, tile_m)
        s_ref[...] = (s_ref[...] * jnp.exp(m_old - m_new)
                      + jnp.sum(jnp.exp(l2 - m_new), axis=1, keepdims=True))
        m_ref[...] = m_new

    @pl.when(v == nv - 1)
    def _():
        copy = jax.nn.sigmoid(copyl_ref[...])
        smpad = jnp.exp(padl_ref[...] - m_ref[...]) / s_ref[...]
        attn3 = attn_ref[...].reshape(_TGT_LEN, _BATCH, _SRC_LEN)
        padmask = (s2t_ref[...] == _PAD_COL).astype(jnp.float32)
        attn_sum = jnp.sum(attn3, axis=2).reshape(_ROWS, 1)
        pad_sum = jnp.sum(attn3 * padmask[None, :, :], axis=2).reshape(_ROWS, 1)
        norm_ref[...] = ((1.0 - copy) * (1.0 - smpad)
                         + copy * (attn_sum - pad_sum) + 1e-10)


def _pass2_body(logits_ref, m_ref, s_ref, copyl_ref, norm_ref, attn_ref,
                s2t_ref, out_ref):
    v = pl.program_id(0)
    copy = jax.nn.sigmoid(copyl_ref[...])
    lt = logits_ref[...].astype(jnp.float32)
    p = (jnp.exp(lt - m_ref[...]) / s_ref[...]) * (1.0 - copy)
    attnw3 = (attn_ref[...] * copy).reshape(_TGT_LEN, _BATCH, _SRC_LEN)
    base = v * _TV
    cols = base + lax.broadcasted_iota(jnp.int32, (_SRC_LEN, _TV), 1)
    parts = []
    for b in range(_BATCH):
        idxb = s2t_ref[b, :]
        oh = (idxb[:, None] == cols).astype(jnp.float32)
        sc_b = jnp.dot(attnw3[:, b, :], oh,
                       preferred_element_type=jnp.float32)
        parts.append(sc_b[:, None, :])
    sc = jnp.concatenate(parts, axis=1).reshape(_ROWS, _TV)
    x = p + sc
    gcol = base + lax.broadcasted_iota(jnp.int32, (1, _TV), 1)
    x = jnp.where(gcol == _PAD_COL, jnp.float32(1e-10), x)
    out_ref[...] = jnp.log(x / norm_ref[...] + 1e-10)


@jax.jit
def kernel(hidden, src, attn, W, b, alignment):
    tgt_vocab = W.shape[0]
    nv = (tgt_vocab + _TV - 1) // _TV
    w_p = W
    b_p = b[None, :]

    words = src[:, :, 0]
    s2t = _sc_gather(alignment, words.T.reshape(-1))
    s2t = s2t.reshape(_BATCH, _SRC_LEN)

    row = pl.BlockSpec((_ROWS, 1), lambda v: (0, 0))
    logits, m, s, copyl, padl, norm = pl.pallas_call(
        functools.partial(_pass1_body, nv, tgt_vocab),
        grid=(nv,),
        in_specs=[
            pl.BlockSpec((_ROWS, _RNN), lambda v: (0, 0)),
            pl.BlockSpec((_TV, _RNN), lambda v: (v, 0)),
            pl.BlockSpec((1, _TV), lambda v: (0, v)),
            pl.BlockSpec((_ROWS, _SRC_LEN), lambda v: (0, 0)),
            pl.BlockSpec((_BATCH, _SRC_LEN), lambda v: (0, 0)),
        ],
        out_specs=[
            pl.BlockSpec((_ROWS, _TV), lambda v: (0, v)),
            row, row, row, row, row,
        ],
        out_shape=[
            jax.ShapeDtypeStruct((_ROWS, tgt_vocab), jnp.bfloat16),
            jax.ShapeDtypeStruct((_ROWS, 1), jnp.float32),
            jax.ShapeDtypeStruct((_ROWS, 1), jnp.float32),
            jax.ShapeDtypeStruct((_ROWS, 1), jnp.float32),
            jax.ShapeDtypeStruct((_ROWS, 1), jnp.float32),
            jax.ShapeDtypeStruct((_ROWS, 1), jnp.float32),
        ],
    )(hidden, w_p, b_p, attn, s2t)

    out = pl.pallas_call(
        _pass2_body,
        grid=(nv,),
        in_specs=[
            pl.BlockSpec((_ROWS, _TV), lambda v: (0, v)),
            row, row, row, row,
            pl.BlockSpec((_ROWS, _SRC_LEN), lambda v: (0, 0)),
            pl.BlockSpec((_BATCH, _SRC_LEN), lambda v: (0, 0)),
        ],
        out_specs=pl.BlockSpec((_ROWS, _TV), lambda v: (0, v)),
        out_shape=jax.ShapeDtypeStruct((_ROWS, tgt_vocab), jnp.float32),
    )(logits, m, s, copyl, norm, attn, s2t)

    return out
